# baseline (speedup 1.0000x reference)
import jax
import jax.numpy as jnp
from jax.experimental import pallas as pl
from jax.experimental.pallas import tpu as pltpu

_N = 8388608
_ROWS = 8192
_COLS = 1024
_CNT_BLK = 512
_OUT_ROWS = _N // 128
_OUT_BLK = 2048


def _count_kernel(x_ref, maj_ref, acc_ref):
    i = pl.program_id(0)

    @pl.when(i == 0)
    def _():
        acc_ref[0] = 0

    acc_ref[0] += jnp.sum(x_ref[...])

    @pl.when(i == pl.num_programs(0) - 1)
    def _():
        maj_ref[0] = (2 * acc_ref[0] > _N).astype(jnp.int32)


def _bcast_kernel(maj_ref, o_ref):
    col = jax.lax.broadcasted_iota(jnp.int32, o_ref.shape, 1)
    o_ref[...] = ((col & 1) == maj_ref[0]).astype(jnp.float32)


def kernel(x):
    maj = pl.pallas_call(
        _count_kernel,
        grid=(_ROWS // _CNT_BLK,),
        in_specs=[pl.BlockSpec((_CNT_BLK, _COLS), lambda i: (i, 0))],
        out_specs=pl.BlockSpec(memory_space=pltpu.SMEM),
        out_shape=jax.ShapeDtypeStruct((1,), jnp.int32),
        scratch_shapes=[pltpu.SMEM((1,), jnp.int32)],
    )(x.reshape(_ROWS, _COLS))

    pred = pl.pallas_call(
        _bcast_kernel,
        grid=(2 * _OUT_ROWS // _OUT_BLK,),
        in_specs=[pl.BlockSpec(memory_space=pltpu.SMEM)],
        out_specs=pl.BlockSpec((_OUT_BLK, 128), lambda i: (i, 0)),
        out_shape=jax.ShapeDtypeStruct((2 * _OUT_ROWS, 128), jnp.float32),
    )(maj)

    return pred.reshape(_N, 2)


# speedup vs baseline: 102.8839x; 102.8839x over previous
import jax
import jax.numpy as jnp
from jax.experimental import pallas as pl
from jax.experimental.pallas import tpu as pltpu

_N = 8388608
_CNT_BLK = 524288
_OUT_BLK = 2048


def _count_kernel(x_ref, maj_ref, acc_ref):
    i = pl.program_id(0)

    @pl.when(i == 0)
    def _():
        acc_ref[0] = 0

    acc_ref[0] += jnp.sum(x_ref[...])

    @pl.when(i == pl.num_programs(0) - 1)
    def _():
        maj_ref[0] = (2 * acc_ref[0] > _N).astype(jnp.int32)


def _bcast_kernel(maj_ref, o_ref):
    c = jax.lax.broadcasted_iota(jnp.int32, o_ref.shape, 1)
    o_ref[...] = (c == maj_ref[0]).astype(jnp.float32)


def kernel(x):
    maj = pl.pallas_call(
        _count_kernel,
        grid=(_N // _CNT_BLK,),
        in_specs=[pl.BlockSpec((_CNT_BLK,), lambda i: (i,))],
        out_specs=pl.BlockSpec(memory_space=pltpu.SMEM),
        out_shape=jax.ShapeDtypeStruct((1,), jnp.int32),
        scratch_shapes=[pltpu.SMEM((1,), jnp.int32)],
    )(x)

    groups = _N // 128
    pred = pl.pallas_call(
        _bcast_kernel,
        grid=(groups // _OUT_BLK,),
        in_specs=[pl.BlockSpec(memory_space=pltpu.SMEM)],
        out_specs=pl.BlockSpec((_OUT_BLK, 2, 128), lambda i: (i, 0, 0)),
        out_shape=jax.ShapeDtypeStruct((groups, 2, 128), jnp.float32),
    )(maj)

    return pred.transpose(0, 2, 1).reshape(_N, 2)
